# baseline (speedup 1.0000x reference)
import jax
import jax.numpy as jnp
from jax.experimental import pallas as pl
from jax.experimental.pallas import tpu as pltpu

_BLOCK_B = 1024


def _mlp_kernel(x_ref, w1_ref, b1_ref, w2_ref, b2_ref, w3_ref, b3_ref,
                w4_ref, b4_ref, o_ref):
    x = x_ref[...]
    h = jnp.tanh(jnp.dot(x, w1_ref[...],
                         preferred_element_type=jnp.float32) + b1_ref[...])
    h = jnp.tanh(jnp.dot(h, w2_ref[...],
                         preferred_element_type=jnp.float32) + b2_ref[...])
    h = jnp.tanh(jnp.dot(h, w3_ref[...],
                         preferred_element_type=jnp.float32) + b3_ref[...])
    o_ref[...] = jnp.dot(h, w4_ref[...],
                         preferred_element_type=jnp.float32) + b4_ref[...]


def kernel(inputs, W1, b1, W2, b2, W3, b3, W4, b4):
    B, d_in = inputs.shape
    H = W1.shape[1]
    nb = B // _BLOCK_B

    b1r = b1.reshape(1, H)
    b2r = b2.reshape(1, H)
    b3r = b3.reshape(1, H)
    b4r = b4.reshape(1, 1)

    full = lambda shape: pl.BlockSpec(shape, lambda i: (0, 0))
    out = pl.pallas_call(
        _mlp_kernel,
        grid=(nb,),
        in_specs=[
            pl.BlockSpec((_BLOCK_B, d_in), lambda i: (i, 0)),
            full(W1.shape), full(b1r.shape),
            full(W2.shape), full(b2r.shape),
            full(W3.shape), full(b3r.shape),
            full(W4.shape), full(b4r.shape),
        ],
        out_specs=pl.BlockSpec((_BLOCK_B, 1), lambda i: (i, 0)),
        out_shape=jax.ShapeDtypeStruct((B, 1), jnp.float32),
        compiler_params=pltpu.CompilerParams(
            dimension_semantics=("arbitrary",),
        ),
    )(inputs, W1, b1r, W2, b2r, W3, b3r, W4, b4r)
    return out


# speedup vs baseline: 1.6213x; 1.6213x over previous
import jax
import jax.numpy as jnp
from jax.experimental import pallas as pl
from jax.experimental.pallas import tpu as pltpu

_BLOCK_B = 8192


def _mlp_kernel(x_ref, w1_ref, b1_ref, w2_ref, b2_ref, w3_ref, b3_ref,
                w4_ref, b4_ref, o_ref):
    x = x_ref[...]
    h = jnp.tanh(jnp.dot(x, w1_ref[...],
                         preferred_element_type=jnp.float32) + b1_ref[...])
    h = jnp.tanh(jnp.dot(h, w2_ref[...],
                         preferred_element_type=jnp.float32) + b2_ref[...])
    h = jnp.tanh(jnp.dot(h, w3_ref[...],
                         preferred_element_type=jnp.float32) + b3_ref[...])
    o_ref[...] = jnp.dot(h, w4_ref[...],
                         preferred_element_type=jnp.float32) + b4_ref[...]


def kernel(inputs, W1, b1, W2, b2, W3, b3, W4, b4):
    B, d_in = inputs.shape
    H = W1.shape[1]
    nb = B // _BLOCK_B

    b1r = b1.reshape(1, H)
    b2r = b2.reshape(1, H)
    b3r = b3.reshape(1, H)
    b4r = b4.reshape(1, 1)

    full = lambda shape: pl.BlockSpec(shape, lambda i: (0, 0))
    out = pl.pallas_call(
        _mlp_kernel,
        grid=(nb,),
        in_specs=[
            pl.BlockSpec((_BLOCK_B, d_in), lambda i: (i, 0)),
            full(W1.shape), full(b1r.shape),
            full(W2.shape), full(b2r.shape),
            full(W3.shape), full(b3r.shape),
            full(W4.shape), full(b4r.shape),
        ],
        out_specs=pl.BlockSpec((_BLOCK_B, 1), lambda i: (i, 0)),
        out_shape=jax.ShapeDtypeStruct((B, 1), jnp.float32),
        compiler_params=pltpu.CompilerParams(
            dimension_semantics=("arbitrary",),
        ),
    )(inputs, W1, b1r, W2, b2r, W3, b3r, W4, b4r)
    return out


# speedup vs baseline: 1.6722x; 1.0314x over previous
import jax
import jax.numpy as jnp
from jax.experimental import pallas as pl
from jax.experimental.pallas import tpu as pltpu

_BLOCK_B = 16384


def _mlp_kernel(x_ref, w1_ref, b1_ref, w2_ref, b2_ref, w3_ref, b3_ref,
                w4_ref, b4_ref, o_ref):
    x = x_ref[...]
    h = jnp.tanh(jnp.dot(x, w1_ref[...],
                         preferred_element_type=jnp.float32) + b1_ref[...])
    h = jnp.tanh(jnp.dot(h, w2_ref[...],
                         preferred_element_type=jnp.float32) + b2_ref[...])
    h = jnp.tanh(jnp.dot(h, w3_ref[...],
                         preferred_element_type=jnp.float32) + b3_ref[...])
    o_ref[...] = jnp.dot(h, w4_ref[...],
                         preferred_element_type=jnp.float32) + b4_ref[...]


def kernel(inputs, W1, b1, W2, b2, W3, b3, W4, b4):
    B, d_in = inputs.shape
    H = W1.shape[1]
    nb = B // _BLOCK_B

    b1r = b1.reshape(1, H)
    b2r = b2.reshape(1, H)
    b3r = b3.reshape(1, H)
    b4r = b4.reshape(1, 1)

    full = lambda shape: pl.BlockSpec(shape, lambda i: (0, 0))
    out = pl.pallas_call(
        _mlp_kernel,
        grid=(nb,),
        in_specs=[
            pl.BlockSpec((_BLOCK_B, d_in), lambda i: (i, 0)),
            full(W1.shape), full(b1r.shape),
            full(W2.shape), full(b2r.shape),
            full(W3.shape), full(b3r.shape),
            full(W4.shape), full(b4r.shape),
        ],
        out_specs=pl.BlockSpec((_BLOCK_B, 1), lambda i: (i, 0)),
        out_shape=jax.ShapeDtypeStruct((B, 1), jnp.float32),
        compiler_params=pltpu.CompilerParams(
            dimension_semantics=("arbitrary",),
        ),
    )(inputs, W1, b1r, W2, b2r, W3, b3r, W4, b4r)
    return out


# speedup vs baseline: 1.6736x; 1.0008x over previous
import jax
import jax.numpy as jnp
from jax.experimental import pallas as pl
from jax.experimental.pallas import tpu as pltpu

_BLOCK_B = 16384


def _mlp_kernel(x_ref, w1_ref, b1_ref, w2_ref, b2_ref, w3_ref, b3_ref,
                w4_ref, b4_ref, o_ref):
    x = x_ref[...]
    h = jnp.tanh(jnp.dot(x, w1_ref[...],
                         preferred_element_type=jnp.float32) + b1_ref[...])
    h = jnp.tanh(jnp.dot(h, w2_ref[...],
                         preferred_element_type=jnp.float32) + b2_ref[...])
    h = jnp.tanh(jnp.dot(h, w3_ref[...],
                         preferred_element_type=jnp.float32) + b3_ref[...])
    o_ref[...] = jnp.dot(h, w4_ref[...],
                         preferred_element_type=jnp.float32) + b4_ref[...]


def kernel(inputs, W1, b1, W2, b2, W3, b3, W4, b4):
    B, d_in = inputs.shape
    H = W1.shape[1]
    nb = B // _BLOCK_B

    b1r = b1.reshape(1, H)
    b2r = b2.reshape(1, H)
    b3r = b3.reshape(1, H)
    b4r = b4.reshape(1, 1)

    full = lambda shape: pl.BlockSpec(shape, lambda i: (0, 0))
    out = pl.pallas_call(
        _mlp_kernel,
        grid=(nb,),
        in_specs=[
            pl.BlockSpec((_BLOCK_B, d_in), lambda i: (i, 0)),
            full(W1.shape), full(b1r.shape),
            full(W2.shape), full(b2r.shape),
            full(W3.shape), full(b3r.shape),
            full(W4.shape), full(b4r.shape),
        ],
        out_specs=pl.BlockSpec((_BLOCK_B, 1), lambda i: (i, 0)),
        out_shape=jax.ShapeDtypeStruct((B, 1), jnp.float32),
        compiler_params=pltpu.CompilerParams(
            dimension_semantics=("parallel",),
        ),
    )(inputs, W1, b1r, W2, b2r, W3, b3r, W4, b4r)
    return out
